# Initial kernel scaffold; baseline (speedup 1.0000x reference)
import jax
import jax.numpy as jnp
from jax.experimental import pallas as pl


def kernel(x_embed, cls_features, prompt, prompt_key):
    raise NotImplementedError("write your pallas kernel here")


# speedup vs baseline: 1.5596x; 1.5596x over previous
import jax
import jax.numpy as jnp
from jax.experimental import pallas as pl


_P = 64
_K = 2
_L = 16
_D = 768
_B = 1024
_BB = 128


def _route_kernel(cls_ref, pk_ref, sim_ref, idx_ref, w_ref, rs_ref):
    pk = pk_ref[...]
    pk_norm = pk * jax.lax.rsqrt(
        jnp.maximum(jnp.sum(pk * pk, axis=1, keepdims=True), 1e-12))
    xc = cls_ref[...]
    x_norm = xc * jax.lax.rsqrt(
        jnp.maximum(jnp.sum(xc * xc, axis=1, keepdims=True), 1e-12))
    sim = jnp.dot(x_norm, pk_norm.T, preferred_element_type=jnp.float32)
    sim_ref[...] = sim

    col = jax.lax.broadcasted_iota(jnp.int32, sim.shape, 1)
    i1 = jnp.argmax(sim, axis=1).astype(jnp.int32)
    v1 = jnp.max(sim, axis=1)
    oh1 = col == i1[:, None]
    sim2 = jnp.where(oh1, -jnp.inf, sim)
    i2 = jnp.argmax(sim2, axis=1).astype(jnp.int32)
    v2 = jnp.max(sim2, axis=1)
    oh2 = col == i2[:, None]

    idx_ref[...] = jnp.concatenate([i1[:, None], i2[:, None]], axis=1)
    w_ref[...] = 0.5 * (oh1.astype(jnp.float32) + oh2.astype(jnp.float32))
    rs_ref[...] = (jnp.sum(v1 + v2) / jnp.float32(_B)).reshape(1, 1)


def _mix_kernel(w_ref, pf_ref, x_ref, out_ref):
    out_ref[...] = jnp.dot(
        w_ref[...], pf_ref[...], preferred_element_type=jnp.float32
    ) + x_ref[...]


def kernel(x_embed, cls_features, prompt, prompt_key):
    x_flat = x_embed.reshape(_B, _L * _D)
    prompt_flat = prompt.reshape(_P, _L * _D)

    sim, idx, w, rs = pl.pallas_call(
        _route_kernel,
        out_shape=(
            jax.ShapeDtypeStruct((_B, _P), jnp.float32),
            jax.ShapeDtypeStruct((_B, _K), jnp.int32),
            jax.ShapeDtypeStruct((_B, _P), jnp.float32),
            jax.ShapeDtypeStruct((1, 1), jnp.float32),
        ),
    )(cls_features, prompt_key)

    nb = _B // _BB
    out_flat = pl.pallas_call(
        _mix_kernel,
        grid=(nb,),
        in_specs=[
            pl.BlockSpec((_BB, _P), lambda i: (i, 0)),
            pl.BlockSpec((_P, _L * _D), lambda i: (0, 0)),
            pl.BlockSpec((_BB, _L * _D), lambda i: (i, 0)),
        ],
        out_specs=pl.BlockSpec((_BB, _L * _D), lambda i: (i, 0)),
        out_shape=jax.ShapeDtypeStruct((_B, _L * _D), jnp.float32),
    )(w, prompt_flat, x_flat)

    prompted = out_flat.reshape(_B, _L, _D)
    return prompted, rs[0, 0], sim, idx


# speedup vs baseline: 3.0066x; 1.9278x over previous
import jax
import jax.numpy as jnp
from jax.experimental import pallas as pl


_P = 64
_K = 2
_L = 16
_D = 768
_B = 1024
_BB = 128


def _route_kernel(cls_ref, pk_ref, sim_ref, idx_ref, w_ref, rs_ref):
    pk = pk_ref[...]
    pk_norm = pk * jax.lax.rsqrt(
        jnp.maximum(jnp.sum(pk * pk, axis=1, keepdims=True), 1e-12))
    xc = cls_ref[...]
    x_norm = xc * jax.lax.rsqrt(
        jnp.maximum(jnp.sum(xc * xc, axis=1, keepdims=True), 1e-12))
    sim = jnp.dot(x_norm, pk_norm.T, preferred_element_type=jnp.float32)
    sim_ref[...] = sim

    col = jax.lax.broadcasted_iota(jnp.int32, sim.shape, 1)
    i1 = jnp.argmax(sim, axis=1).astype(jnp.int32)
    v1 = jnp.max(sim, axis=1)
    oh1 = col == i1[:, None]
    sim2 = jnp.where(oh1, -jnp.inf, sim)
    i2 = jnp.argmax(sim2, axis=1).astype(jnp.int32)
    v2 = jnp.max(sim2, axis=1)
    oh2 = col == i2[:, None]

    idx_ref[...] = jnp.concatenate([i1[:, None], i2[:, None]], axis=1)
    w_ref[...] = 0.5 * (oh1.astype(jnp.float32) + oh2.astype(jnp.float32))
    rs_ref[...] = (jnp.sum(v1 + v2) / jnp.float32(_B)).reshape(1, 1)


def _mix_kernel(w_ref, pf_ref, x_ref, out_ref):
    w = w_ref[...]
    for l in range(_L):
        out_ref[:, l, :] = jnp.dot(
            w, pf_ref[:, l, :], preferred_element_type=jnp.float32
        ) + x_ref[:, l, :]


def kernel(x_embed, cls_features, prompt, prompt_key):
    sim, idx, w, rs = pl.pallas_call(
        _route_kernel,
        out_shape=(
            jax.ShapeDtypeStruct((_B, _P), jnp.float32),
            jax.ShapeDtypeStruct((_B, _K), jnp.int32),
            jax.ShapeDtypeStruct((_B, _P), jnp.float32),
            jax.ShapeDtypeStruct((1, 1), jnp.float32),
        ),
    )(cls_features, prompt_key)

    nb = _B // _BB
    prompted = pl.pallas_call(
        _mix_kernel,
        grid=(nb,),
        in_specs=[
            pl.BlockSpec((_BB, _P), lambda i: (i, 0)),
            pl.BlockSpec((_P, _L, _D), lambda i: (0, 0, 0)),
            pl.BlockSpec((_BB, _L, _D), lambda i: (i, 0, 0)),
        ],
        out_specs=pl.BlockSpec((_BB, _L, _D), lambda i: (i, 0, 0)),
        out_shape=jax.ShapeDtypeStruct((_B, _L, _D), jnp.float32),
    )(w, prompt, x_embed)

    return prompted, rs[0, 0], sim, idx
